# baseline (device time: 46360 ns/iter reference)
import jax
import jax.numpy as jnp
from jax import lax
from jax.experimental import pallas as pl
from jax.experimental.pallas import tpu as pltpu

N_DEV = 4
EPS = 1e-5
BM = 512
D = 2
P = 2
RING = 6
NOB = 3


def kernel(x, gamma, beta):
    m, n_loc = x.shape
    n_glob = N_DEV * n_loc
    nb = m // BM

    def body(x_hbm, g_ref, b_ref, out_hbm, *refs):
        xbufs = refs[:RING]
        obufs = refs[RING:RING + NOB]
        comm_ref = refs[RING + NOB]
        xsems, osems, send_sems, recv_sems = refs[RING + NOB + 1:]
        my = lax.axis_index("i")

        def fetch(blk):
            pltpu.make_async_copy(
                x_hbm.at[pl.ds(blk * BM, BM), :],
                xbufs[blk % RING],
                xsems.at[blk % RING],
            ).start()

        def fetch_wait(blk):
            pltpu.make_async_copy(
                x_hbm.at[pl.ds(blk * BM, BM), :],
                xbufs[blk % RING],
                xsems.at[blk % RING],
            ).wait()

        def store(j):
            pltpu.make_async_copy(
                obufs[j % NOB],
                out_hbm.at[pl.ds(j * BM, BM), :],
                osems.at[j % NOB],
            ).start()

        def store_wait(j):
            pltpu.make_async_copy(
                obufs[j % NOB],
                out_hbm.at[pl.ds(j * BM, BM), :],
                osems.at[j % NOB],
            ).wait()

        def rdma_for(blk, off):
            return pltpu.make_async_remote_copy(
                src_ref=comm_ref.at[0, :, pl.ds(blk * BM, BM)],
                dst_ref=comm_ref.at[off, :, pl.ds(blk * BM, BM)],
                send_sem=send_sems.at[blk, off],
                recv_sem=recv_sems.at[blk, off],
                device_id=lax.rem(my + off, N_DEV),
                device_id_type=pl.DeviceIdType.LOGICAL,
            )

        def drain(j):
            for off in range(1, N_DEV):
                rdma_for(j, off).wait_recv()
            tot = jnp.sum(comm_ref[:, :, pl.ds(j * BM, BM)], axis=0)
            mean = tot[0:1, :] * (1.0 / n_glob)
            ex2 = tot[1:2, :] * (1.0 / n_glob)
            rstd = lax.rsqrt(ex2 - mean * mean + EPS)
            mean_c = mean.reshape(BM, 1)
            rstd_c = rstd.reshape(BM, 1)
            if j >= NOB:
                store_wait(j - NOB)
            xf = xbufs[j % RING][:, :]
            obufs[j % NOB][:, :] = (
                g_ref[:, :] * ((xf - mean_c) * rstd_c) + b_ref[:, :]
            ).astype(jnp.bfloat16)
            store(j)

        barrier = pltpu.get_barrier_semaphore()
        for off in range(1, N_DEV):
            pl.semaphore_signal(
                barrier, inc=1,
                device_id=lax.rem(my + off, N_DEV),
                device_id_type=pl.DeviceIdType.LOGICAL,
            )
        pl.semaphore_wait(barrier, N_DEV - 1)

        sends = []
        for k in range(min(P + 1, nb)):
            fetch(k)
        for blk in range(nb):
            nxt = blk + P + 1
            if nxt < nb:
                fetch(nxt)
            fetch_wait(blk)
            xf = xbufs[blk % RING][:, :]
            ps = jnp.sum(xf, axis=1)
            pss = jnp.sum(xf * xf, axis=1)
            comm_ref[0, :, pl.ds(blk * BM, BM)] = jnp.stack([ps, pss], axis=0)
            for off in range(1, N_DEV):
                r = rdma_for(blk, off)
                r.start()
                sends.append(r)
            if blk >= D:
                drain(blk - D)
        for j in range(nb - D, nb):
            drain(j)

        for r in sends:
            r.wait_send()
        for j in range(nb - NOB, nb):
            store_wait(j)

    return pl.pallas_call(
        body,
        out_shape=jax.ShapeDtypeStruct((m, n_loc), jnp.bfloat16),
        in_specs=[
            pl.BlockSpec(memory_space=pl.ANY),
            pl.BlockSpec(memory_space=pltpu.VMEM),
            pl.BlockSpec(memory_space=pltpu.VMEM),
        ],
        out_specs=pl.BlockSpec(memory_space=pl.ANY),
        scratch_shapes=(
            [pltpu.VMEM((BM, n_loc), jnp.float32) for _ in range(RING)]
            + [pltpu.VMEM((BM, n_loc), jnp.bfloat16) for _ in range(NOB)]
            + [
                pltpu.VMEM((N_DEV, 2, m), jnp.float32),
                pltpu.SemaphoreType.DMA((RING,)),
                pltpu.SemaphoreType.DMA((NOB,)),
                pltpu.SemaphoreType.DMA((m // BM, N_DEV)),
                pltpu.SemaphoreType.DMA((m // BM, N_DEV)),
            ]
        ),
        compiler_params=pltpu.CompilerParams(
            collective_id=0, vmem_limit_bytes=100 * 1024 * 1024
        ),
    )(x, gamma.reshape(1, n_loc), beta.reshape(1, n_loc))
